# baseline (speedup 1.0000x reference)
import functools

import jax
import jax.numpy as jnp
from jax import lax
from jax.experimental import pallas as pl
from jax.experimental.pallas import tpu as pltpu
from jax.experimental.pallas import tpu_sc as plsc

_P = 32
_HW = 64
_CHUNK = 128


def _sc_gather(B, Bs, C, part):
    mesh = plsc.VectorSubcoreMesh(core_axis_name="c", subcore_axis_name="s")
    info = plsc.get_sparse_core_info()
    nw = info.num_cores * info.num_subcores
    per_w = Bs // nw
    n_chunks = per_w // _CHUNK
    CHW = C * _HW

    @functools.partial(
        pl.kernel,
        mesh=mesh,
        compiler_params=pltpu.CompilerParams(needs_layout_passes=False),
        out_type=jax.ShapeDtypeStruct((Bs, C * _P), jnp.float32),
        scratch_types=[
            pltpu.VMEM((_HW, _CHUNK), jnp.int32),
            pltpu.VMEM((_CHUNK + 1, CHW), jnp.float32),
            pltpu.VMEM((16 * 40,), jnp.int32),
            pltpu.VMEM((_CHUNK // 2, C * _P), jnp.float32),
        ],
    )
    def k(ids_hbm, board_hbm, raw_hbm, ids_v, board_v, fidx_v, out_v):
        wid = lax.axis_index("s") * info.num_cores + lax.axis_index("c")
        base_rel = wid * per_w
        base = part * Bs + base_rel
        lane = lax.iota(jnp.int32, 16)
        sent = jnp.full((16,), _HW, jnp.int32)

        lane40 = lane * 40
        zero16 = jnp.zeros((16,), jnp.float32)
        for j in range(CHW // 16):
            board_v[_CHUNK, pl.ds(j * 16, 16)] = zero16

        def chunk_body(ch, carry):
            s0 = base + ch * _CHUNK
            s0r = base_rel + ch * _CHUNK
            pltpu.sync_copy(ids_hbm.at[:, pl.ds(s0, _CHUNK)], ids_v)
            pltpu.sync_copy(board_hbm.at[pl.ds(s0, _CHUNK)], board_v.at[pl.ds(0, _CHUNK)])
            for sub in range(2):
                for g in range(_CHUNK // 32):
                    gg = sub * (_CHUNK // 32) + g
                    for i in range(16):
                        fidx_v[pl.ds(i * 40, 16)] = sent
                        fidx_v[pl.ds(i * 40 + 16, 16)] = sent

                    def hw_body(t, c2):
                        hw = 63 - t
                        idrow = ids_v[hw, pl.ds(gg * 16, 16)]
                        rowp = jnp.where(idrow > 0, idrow - 1, 32)
                        plsc.store_scatter(
                            fidx_v, [lane40 + rowp],
                            jnp.broadcast_to(hw, (16,)).astype(jnp.int32),
                        )
                        return c2

                    lax.fori_loop(0, _HW, hw_body, 0)

                    def s_body(si, c2):
                        s = g * 16 + si
                        srow = jnp.broadcast_to(
                            sub * (_CHUNK // 2) + s, (16,)
                        ).astype(jnp.int32)
                        for half in range(2):
                            fp = fidx_v[pl.ds(si * 40 + half * 16, 16)]
                            valid = fp < _HW
                            srow_v = jnp.where(valid, srow, _CHUNK)
                            fp0 = jnp.where(valid, fp, 0)
                            for c in range(C):
                                vals = plsc.load_gather(
                                    board_v, [srow_v, fp0 + c * _HW]
                                )
                                out_v[s, pl.ds(c * _P + half * 16, 16)] = vals
                        return c2

                    lax.fori_loop(0, 16, s_body, 0)
                pltpu.sync_copy(
                    out_v, raw_hbm.at[pl.ds(s0r + sub * (_CHUNK // 2), _CHUNK // 2)]
                )
            return carry

        lax.fori_loop(0, n_chunks, chunk_body, 0)

    return k


def _proj_body(raw_ref, bigw_ref, bias_ref, out_ref):
    out_ref[...] = (
        jnp.dot(raw_ref[...], bigw_ref[...], preferred_element_type=jnp.float32)
        + bias_ref[...]
    )


def _proj_body_acc(raw_ref, bigw_ref, bias_ref, obuf_ref, out_ref):
    del obuf_ref
    out_ref[...] = (
        jnp.dot(raw_ref[...], bigw_ref[...], preferred_element_type=jnp.float32)
        + bias_ref[...]
    )


def kernel(full_board_vector, piece_ids, Wm, bv):
    B, C, H, W = full_board_vector.shape
    HW = H * W
    F = Wm.shape[0]
    P = _P
    ids_t = piece_ids.reshape(B, HW).astype(jnp.int32).T
    board = full_board_vector.reshape(B, C * HW)

    S = 2
    Bs = B // S
    raws = [_sc_gather(B, Bs, C, p)(ids_t, board) for p in range(S)]

    eye = jnp.eye(P, dtype=jnp.float32)
    bigw = jnp.einsum("pq,fc->cpqf", eye, Wm).reshape(C * P, P * F)
    bias = jnp.tile(bv, P).reshape(1, P * F)

    BBLK = 2048
    nblk = Bs // BBLK
    out = pl.pallas_call(
        _proj_body,
        grid=(nblk,),
        in_specs=[
            pl.BlockSpec((BBLK, C * P), lambda i: (i, 0)),
            pl.BlockSpec((C * P, P * F), lambda i: (0, 0)),
            pl.BlockSpec((1, P * F), lambda i: (0, 0)),
        ],
        out_specs=pl.BlockSpec((BBLK, P * F), lambda i: (i, 0)),
        out_shape=jax.ShapeDtypeStruct((B, P * F), jnp.float32),
    )(raws[0], bigw, bias)
    for p in range(1, S):
        out = pl.pallas_call(
            _proj_body_acc,
            grid=(nblk,),
            in_specs=[
                pl.BlockSpec((BBLK, C * P), lambda i: (i, 0)),
                pl.BlockSpec((C * P, P * F), lambda i: (0, 0)),
                pl.BlockSpec((1, P * F), lambda i: (0, 0)),
                pl.BlockSpec(memory_space=pl.ANY),
            ],
            out_specs=pl.BlockSpec(
                (BBLK, P * F), lambda i, p=p: (i + p * nblk, 0)
            ),
            out_shape=jax.ShapeDtypeStruct((B, P * F), jnp.float32),
            input_output_aliases={3: 0},
        )(raws[p], bigw, bias, out)
    return out.reshape(B, P, F)


# speedup vs baseline: 1.0736x; 1.0736x over previous
import functools

import jax
import jax.numpy as jnp
from jax import lax
from jax.experimental import pallas as pl
from jax.experimental.pallas import tpu as pltpu
from jax.experimental.pallas import tpu_sc as plsc

_P = 32
_HW = 64
_CHUNK = 64


def _sc_gather(B, Bs, C, part):
    mesh = plsc.VectorSubcoreMesh(core_axis_name="c", subcore_axis_name="s")
    info = plsc.get_sparse_core_info()
    nw = info.num_cores * info.num_subcores
    per_w = Bs // nw
    n_chunks = per_w // _CHUNK
    CHW = C * _HW

    @functools.partial(
        pl.kernel,
        mesh=mesh,
        compiler_params=pltpu.CompilerParams(needs_layout_passes=False),
        out_type=jax.ShapeDtypeStruct((Bs, C * _P), jnp.float32),
        scratch_types=[
            pltpu.VMEM((_HW, 2 * _CHUNK), jnp.int32),
            pltpu.VMEM((_CHUNK + 1, CHW), jnp.float32),
            pltpu.VMEM((_CHUNK + 1, CHW), jnp.float32),
            pltpu.VMEM((16 * 40,), jnp.int32),
            pltpu.VMEM((_CHUNK // 2, C * _P), jnp.float32),
            pltpu.VMEM((_CHUNK // 2, C * _P), jnp.float32),
            pltpu.SemaphoreType.DMA,
            pltpu.SemaphoreType.DMA,
            pltpu.SemaphoreType.DMA,
            pltpu.SemaphoreType.DMA,
        ],
    )
    def k(ids_hbm, board_hbm, raw_hbm, ids_v, bv0, bv1, fidx_v, ov0, ov1,
          sb0, sb1, so0, so1):
        wid = lax.axis_index("s") * info.num_cores + lax.axis_index("c")
        base_rel = wid * per_w
        base = part * Bs + base_rel
        lane = lax.iota(jnp.int32, 16)
        sent = jnp.full((16,), _HW, jnp.int32)
        boards, bsems = [bv0, bv1], [sb0, sb1]
        outs, osems = [ov0, ov1], [so0, so1]

        lane40 = lane * 40
        zero16 = jnp.zeros((16,), jnp.float32)
        for j in range(CHW // 16):
            bv0[_CHUNK, pl.ds(j * 16, 16)] = zero16
            bv1[_CHUNK, pl.ds(j * 16, 16)] = zero16

        def start_board(ch):
            return pltpu.async_copy(
                board_hbm.at[pl.ds(base + ch * _CHUNK, _CHUNK)],
                boards[ch % 2].at[pl.ds(0, _CHUNK)],
                bsems[ch % 2],
            )

        board_dma = [None, None]
        out_dma = [None, None]
        board_dma[0] = start_board(0)
        for ch in range(n_chunks):
            if ch % 2 == 0:
                pltpu.sync_copy(
                    ids_hbm.at[:, pl.ds(base + ch * _CHUNK, 2 * _CHUNK)], ids_v
                )
            if ch + 1 < n_chunks:
                board_dma[(ch + 1) % 2] = start_board(ch + 1)
            board_dma[ch % 2].wait()
            board_v = boards[ch % 2]
            s0r = base_rel + ch * _CHUNK
            for sub in range(2):
                q = 2 * ch + sub
                out_v = outs[q % 2]
                if out_dma[q % 2] is not None:
                    out_dma[q % 2].wait()
                for g in range(_CHUNK // 32):
                    gg = sub * (_CHUNK // 32) + g
                    idcol0 = (ch % 2) * _CHUNK + gg * 16
                    for i in range(16):
                        fidx_v[pl.ds(i * 40, 16)] = sent
                        fidx_v[pl.ds(i * 40 + 16, 16)] = sent

                    def hw_body(t, c2, idcol0=idcol0):
                        hw = 63 - t
                        idrow = ids_v[hw, pl.ds(idcol0, 16)]
                        rowp = jnp.where(idrow > 0, idrow - 1, 32)
                        plsc.store_scatter(
                            fidx_v, [lane40 + rowp],
                            jnp.broadcast_to(hw, (16,)).astype(jnp.int32),
                        )
                        return c2

                    lax.fori_loop(0, _HW, hw_body, 0)

                    def s_body(si, c2, g=g, sub=sub, board_v=board_v,
                               out_v=out_v):
                        s = g * 16 + si
                        srow = jnp.broadcast_to(
                            sub * (_CHUNK // 2) + s, (16,)
                        ).astype(jnp.int32)
                        for half in range(2):
                            fp = fidx_v[pl.ds(si * 40 + half * 16, 16)]
                            valid = fp < _HW
                            srow_v = jnp.where(valid, srow, _CHUNK)
                            fp0 = jnp.where(valid, fp, 0)
                            for c in range(C):
                                vals = plsc.load_gather(
                                    board_v, [srow_v, fp0 + c * _HW]
                                )
                                out_v[s, pl.ds(c * _P + half * 16, 16)] = vals
                        return c2

                    lax.fori_loop(0, 16, s_body, 0)
                out_dma[q % 2] = pltpu.async_copy(
                    out_v,
                    raw_hbm.at[pl.ds(s0r + sub * (_CHUNK // 2), _CHUNK // 2)],
                    osems[q % 2],
                )
        for d in out_dma:
            if d is not None:
                d.wait()

    return k


def _proj_body(raw_ref, bigw_ref, bias_ref, out_ref):
    out_ref[...] = (
        jnp.dot(raw_ref[...], bigw_ref[...], preferred_element_type=jnp.float32)
        + bias_ref[...]
    )


def _proj_body_acc(raw_ref, bigw_ref, bias_ref, obuf_ref, out_ref):
    del obuf_ref
    out_ref[...] = (
        jnp.dot(raw_ref[...], bigw_ref[...], preferred_element_type=jnp.float32)
        + bias_ref[...]
    )


def kernel(full_board_vector, piece_ids, Wm, bv):
    B, C, H, W = full_board_vector.shape
    HW = H * W
    F = Wm.shape[0]
    P = _P
    ids_t = piece_ids.reshape(B, HW).astype(jnp.int32).T
    board = full_board_vector.reshape(B, C * HW)

    S = 2
    Bs = B // S
    raws = [_sc_gather(B, Bs, C, p)(ids_t, board) for p in range(S)]

    eye = jnp.eye(P, dtype=jnp.float32)
    bigw = jnp.einsum("pq,fc->cpqf", eye, Wm).reshape(C * P, P * F)
    bias = jnp.tile(bv, P).reshape(1, P * F)

    BBLK = 2048
    nblk = Bs // BBLK
    out = pl.pallas_call(
        _proj_body,
        grid=(nblk,),
        in_specs=[
            pl.BlockSpec((BBLK, C * P), lambda i: (i, 0)),
            pl.BlockSpec((C * P, P * F), lambda i: (0, 0)),
            pl.BlockSpec((1, P * F), lambda i: (0, 0)),
        ],
        out_specs=pl.BlockSpec((BBLK, P * F), lambda i: (i, 0)),
        out_shape=jax.ShapeDtypeStruct((B, P * F), jnp.float32),
    )(raws[0], bigw, bias)
    for p in range(1, S):
        out = pl.pallas_call(
            _proj_body_acc,
            grid=(nblk,),
            in_specs=[
                pl.BlockSpec((BBLK, C * P), lambda i: (i, 0)),
                pl.BlockSpec((C * P, P * F), lambda i: (0, 0)),
                pl.BlockSpec((1, P * F), lambda i: (0, 0)),
                pl.BlockSpec(memory_space=pl.ANY),
            ],
            out_specs=pl.BlockSpec(
                (BBLK, P * F), lambda i, p=p: (i + p * nblk, 0)
            ),
            out_shape=jax.ShapeDtypeStruct((B, P * F), jnp.float32),
            input_output_aliases={3: 0},
        )(raws[p], bigw, bias, out)
    return out.reshape(B, P, F)
